# Initial kernel scaffold; baseline (speedup 1.0000x reference)
import jax
import jax.numpy as jnp
from jax.experimental import pallas as pl


def kernel(inputs, W0, W1, W2):
    raise NotImplementedError("write your pallas kernel here")


# speedup vs baseline: 1.4926x; 1.4926x over previous
import functools

import jax
import jax.numpy as jnp
from jax import lax
from jax.experimental import pallas as pl
from jax.experimental.pallas import tpu as pltpu
from jax.experimental.pallas import tpu_sc as plsc

VOCAB = 100000
HIDDEN = 768
NUM_TABLES = 3
B = 4 * 2048
NC, NS = 2, 16
NW = NC * NS
B_PER_W = B // NW
CHUNK = 64
NCHUNK = B_PER_W // CHUNK


@jax.jit
def _gather3(idx_flat, W0, W1, W2):
    out = jax.ShapeDtypeStruct((B, HIDDEN), jnp.float32)
    mesh = plsc.VectorSubcoreMesh(core_axis_name="c", subcore_axis_name="s")

    @functools.partial(
        pl.kernel,
        out_type=(out, out, out),
        mesh=mesh,
        scratch_types=[
            *[pltpu.VMEM((CHUNK,), jnp.int32) for _ in range(NCHUNK)],
            pltpu.VMEM((CHUNK, HIDDEN), jnp.float32),
            pltpu.VMEM((CHUNK, HIDDEN), jnp.float32),
            pltpu.SemaphoreType.DMA,
            pltpu.SemaphoreType.DMA,
        ],
    )
    def k(idx_hbm, w0_hbm, w1_hbm, w2_hbm, o0_hbm, o1_hbm, o2_hbm,
          i0, i1, i2, i3, buf0, buf1, sem0, sem1):
        idxs = (i0, i1, i2, i3)
        bufs = (buf0, buf1)
        sems = (sem0, sem1)
        tables = (w0_hbm, w1_hbm, w2_hbm)
        outs = (o0_hbm, o1_hbm, o2_hbm)

        wid = lax.axis_index("s") * NC + lax.axis_index("c")
        base = wid * B_PER_W

        for c in range(NCHUNK):
            pltpu.sync_copy(idx_hbm.at[pl.ds(base + c * CHUNK, CHUNK)], idxs[c])

        items = [(t, c) for t in range(NUM_TABLES) for c in range(NCHUNK)]
        copies = []

        def start(j):
            t, c = items[j]
            copies.append(
                pltpu.async_copy(tables[t].at[idxs[c]], bufs[j % 2], sems[j % 2])
            )

        start(0)
        for j in range(len(items)):
            if j + 1 < len(items):
                start(j + 1)
            copies[j].wait()
            t, c = items[j]
            pltpu.sync_copy(bufs[j % 2], outs[t].at[pl.ds(base + c * CHUNK, CHUNK)])

    return k(idx_flat, W0, W1, W2)


def kernel(inputs, W0, W1, W2):
    idx = inputs.reshape(-1).astype(jnp.int32)
    e0, e1, e2 = _gather3(idx, W0, W1, W2)
    shp = (*inputs.shape, HIDDEN)
    e0 = e0.reshape(shp)
    e1 = e1.reshape(shp)
    e2 = e2.reshape(shp)
    return (e0, e1, e2, e2, e1, e0)
